# baseline (speedup 1.0000x reference)
import functools

import jax
import jax.numpy as jnp
from jax import lax
from jax.experimental import pallas as pl
from jax.experimental.pallas import tpu as pltpu
from jax.experimental.pallas import tpu_sc as plsc

N = 10000
NN = 2 * N
D = 128
E = 320000
P = 20000
CH = 128
IB = 8
EPAD = 327680
ET = EPAD // 16
NCH = ET // CH
NIBP = NCH // (2 * IB)
ACC_R = 10112
DUMMY = N
CPO = 632
OUTR = 16 * CPO
IPAD = 40960
IT = IPAD // 32

@functools.cache
def _mesh():
    return plsc.VectorSubcoreMesh(core_axis_name="c", subcore_axis_name="s")


def _zero_fill(buf, rows, width):
    zv = jnp.zeros((16,), jnp.float32)

    def body(i, _):
        for j in range(width // 16):
            buf[i, pl.ds(j * 16, 16)] = zv
        return 0

    lax.fori_loop(0, rows, body, 0)


def _zero_acc(src_buf, acc, s):
    for j in range(4):
        pltpu.sync_copy(src_buf, acc.at[pl.ds(s * CPO + j * 128, 128)])
    pltpu.sync_copy(src_buf.at[pl.ds(0, CPO - 512)],
                    acc.at[pl.ds(s * CPO + 512, CPO - 512)])


def _segsum_body(x, idx2, out, acc, idxa, r0, r1, sg0, sg1):
    c = lax.axis_index("c")
    s = lax.axis_index("s")

    _zero_fill(r0, CH, D)
    _zero_acc(r0, acc, s)

    plsc.subcore_barrier()

    rowbase = (c * (EPAD // CH) + s * NCH) * 2

    def body(ib, _):
        pltpu.sync_copy(idx2.at[pl.ds(rowbase + 2 * IB * ib, 2 * IB)], idxa)
        for jp in range(IB // 2):
            j0, j1 = 2 * jp, 2 * jp + 1
            ha = pltpu.async_copy(x.at[idxa.at[2 * j0]], r0, sg0)
            hb = pltpu.async_copy(x.at[idxa.at[2 * j1]], r1, sg1)
            ha.wait()
            pltpu.sync_copy(r0, acc.at[idxa.at[2 * j0 + 1]], add=True)
            hb.wait()
            pltpu.sync_copy(r1, acc.at[idxa.at[2 * j1 + 1]], add=True)
        return 0

    lax.fori_loop(0, NCH // IB, body, 0)

    plsc.subcore_barrier()

    pltpu.sync_copy(
        acc.at[pl.ds(s * CPO, CPO)],
        out.at[pl.ds(c * OUTR + s * CPO, CPO)])


@functools.cache
def _segsum():
    return pl.kernel(
    _segsum_body,
    out_type=jax.ShapeDtypeStruct((2 * OUTR, D), jnp.float32),
    mesh=_mesh(),
    scratch_types=[
        pltpu.VMEM_SHARED((ACC_R, D), jnp.float32),
        pltpu.VMEM((2 * IB, CH), jnp.int32),
        pltpu.VMEM((CH, D), jnp.float32),
        pltpu.VMEM((CH, D), jnp.float32),
        pltpu.SemaphoreType.DMA,
        pltpu.SemaphoreType.DMA,
    ],
)


def _deg_body(dst2, outdeg, accd, didx, ones, zbufd):
    c = lax.axis_index("c")
    s = lax.axis_index("s")

    ov = jnp.ones((16,), jnp.float32)
    zv = jnp.zeros((16,), jnp.float32)

    def fill(i, _):
        for j in range(D // 16):
            ones[i, pl.ds(j * 16, 16)] = ov
            zbufd[i, pl.ds(j * 16, 16)] = zv
        return 0

    lax.fori_loop(0, CH, fill, 0)
    _zero_acc(zbufd, accd, s)

    plsc.subcore_barrier()

    ebase = c * EPAD + s * ET

    def echunk(i, _):
        pltpu.sync_copy(dst2.at[pl.ds(ebase + i * CH, CH)], didx)
        pltpu.sync_copy(ones, accd.at[didx], add=True)
        return 0

    lax.fori_loop(0, NCH, echunk, 0)

    plsc.subcore_barrier()

    pltpu.sync_copy(
        accd.at[pl.ds(s * CPO, CPO)],
        outdeg.at[pl.ds(c * OUTR + s * CPO, CPO)])


@functools.cache
def _deg():
    return pl.kernel(
    _deg_body,
    out_type=jax.ShapeDtypeStruct((2 * OUTR, D), jnp.float32),
    mesh=_mesh(),
    scratch_types=[
        pltpu.VMEM_SHARED((ACC_R, D), jnp.float32),
        pltpu.VMEM((CH,), jnp.int32),
        pltpu.VMEM((CH, D), jnp.float32),
        pltpu.VMEM((CH, D), jnp.float32),
    ],
)


def _pair_gather_body(table, idx, out, sidx, rows, sem):
    c = lax.axis_index("c")
    s = lax.axis_index("s")
    wid = s * 2 + c
    base = wid * IT

    def chunk(i, _):
        b0 = base + i * CH
        pltpu.sync_copy(idx.at[pl.ds(b0, CH)], sidx)
        pltpu.async_copy(table.at[sidx], rows, sem).wait()
        pltpu.sync_copy(rows, out.at[pl.ds(b0, CH)])
        return 0

    lax.fori_loop(0, IT // CH, chunk, 0)


@functools.cache
def _pair_gather():
    return pl.kernel(
    _pair_gather_body,
    out_type=jax.ShapeDtypeStruct((IPAD, D), jnp.float32),
    mesh=_mesh(),
    scratch_types=[
        pltpu.VMEM((CH,), jnp.int32),
        pltpu.VMEM((CH, D), jnp.float32),
        pltpu.SemaphoreType.DMA,
    ],
)


BR = 400
NBLK = NN // BR


def _dense_body(x_r, agg_r, deg_r, ws_r, wn_r, b_r, g_r, be_r, h_r, cs_r):
    x = x_r[...]
    agg = agg_r[...] / jnp.maximum(deg_r[...], 1.0)
    h = (jnp.dot(x, ws_r[...], preferred_element_type=jnp.float32)
         + jnp.dot(agg, wn_r[...], preferred_element_type=jnp.float32)
         + b_r[...])
    mu = h.mean(axis=-1, keepdims=True)
    var = h.var(axis=-1, keepdims=True)
    h = (h - mu) / jnp.sqrt(var + 1e-5) * g_r[...] + be_r[...]
    h_r[...] = jax.nn.gelu(h)
    cs_r[...] = jnp.broadcast_to(jnp.sum(x, axis=0, keepdims=True)[None],
                                 (1, 8, D))


@functools.cache
def _dense():
    bs = pl.BlockSpec((BR, D), lambda i: (i, 0))
    wfull = pl.BlockSpec((D, D), lambda i: (0, 0))
    vec = pl.BlockSpec((1, D), lambda i: (0, 0))
    return pl.pallas_call(
        _dense_body,
        grid=(NBLK,),
        in_specs=[bs, bs, pl.BlockSpec((BR, 1), lambda i: (i, 0)),
                  wfull, wfull, vec, vec, vec],
        out_specs=[bs, pl.BlockSpec((1, 8, D), lambda i: (i, 0, 0))],
        out_shape=[jax.ShapeDtypeStruct((NN, D), jnp.float32),
                   jax.ShapeDtypeStruct((NBLK, 8, D), jnp.float32)],
    )


def _jk_body(h1_r, h2_r, h3_r, inz_r, g_r, b_r, wl_r, bl_r, wa_r, wb_r, u_r):
    hs = [h1_r[...], h2_r[...], h3_r[...]]
    tot = sum(h.sum(axis=-1, keepdims=True) for h in hs)
    tot2 = sum((h * h).sum(axis=-1, keepdims=True) for h in hs)
    mu = tot / (3 * D)
    var = tot2 / (3 * D) - mu * mu
    rstd = 1.0 / jnp.sqrt(var + 1e-5)
    z = bl_r[...]
    for j, h in enumerate(hs):
        lnh = (h - mu) * rstd * g_r[j] + b_r[j]
        z = z + jnp.dot(lnh, wl_r[j], preferred_element_type=jnp.float32)
    u_r[...] = (jnp.dot(inz_r[...], wa_r[0],
                        preferred_element_type=jnp.float32)
                + jnp.dot(z, wb_r[0], preferred_element_type=jnp.float32))


@functools.cache
def _jk():
    bs = pl.BlockSpec((BR, D), lambda i: (i, 0))
    w3 = pl.BlockSpec((3, D, D), lambda i: (0, 0, 0))
    v3 = pl.BlockSpec((3, 1, D), lambda i: (0, 0, 0))
    vec = pl.BlockSpec((1, D), lambda i: (0, 0))
    wg = pl.BlockSpec((1, D, D), lambda i: (i // (NBLK // 2), 0, 0))
    return pl.pallas_call(
        _jk_body,
        grid=(NBLK,),
        in_specs=[bs, bs, bs, bs, v3, v3, w3, vec, wg, wg],
        out_specs=bs,
        out_shape=jax.ShapeDtypeStruct((NN, D), jnp.float32),
    )


def _contact_body2(p1_r, p2_r, bc1_r, w2_r, b2_r, w3_r, b3_r, w4_r, bc4_r,
                   ep_r):
    h = jax.nn.gelu(p1_r[...] + p2_r[...] + bc1_r[...])
    h = jax.nn.gelu(jnp.dot(h, w2_r[...], preferred_element_type=jnp.float32)
                    + b2_r[...])
    h = jax.nn.gelu(jnp.dot(h, w3_r[...], preferred_element_type=jnp.float32)
                    + b3_r[...])
    mu = h.mean(axis=-1, keepdims=True)
    var = h.var(axis=-1, keepdims=True)
    h = jnp.tanh((h - mu) / jnp.sqrt(var + 1e-5))
    logit = jnp.sum(h * w4_r[...], axis=-1, keepdims=True) + bc4_r[0, 0]
    ep_r[...] = jax.nn.sigmoid(logit)


PBLK = P // BR


@functools.cache
def _contact():
    bs = pl.BlockSpec((BR, D), lambda i: (i, 0))
    wfull = pl.BlockSpec((D, D), lambda i: (0, 0))
    vec = pl.BlockSpec((1, D), lambda i: (0, 0))
    one = pl.BlockSpec((1, 1), lambda i: (0, 0))
    bs2 = pl.BlockSpec((BR, D), lambda i: (i + PBLK, 0))
    return pl.pallas_call(
        _contact_body2,
        grid=(PBLK,),
        in_specs=[bs, bs2, vec, wfull, vec, wfull, vec, vec, one],
        out_specs=pl.BlockSpec((BR, 1), lambda i: (i, 0)),
        out_shape=jax.ShapeDtypeStruct((P, 1), jnp.float32),
    )


def _god_body(xg_r, m_r, wsg_r, wng_r, bg_r, gg_r, beg_r,
              w1_r, b1_r, w2_r, b2_r, w3_r, b3_r, w4_r, b4_r,
              lng_r, lnb_r, wp1_r, bp1_r, wp2_r, bp2_r, wp3_r, bp3_r,
              ln2g_r, ln2b_r, wp4_r, bp4_r,
              g_out, ip_out, fx_out):
    def ln(x, g, b):
        mu = x.mean(axis=-1, keepdims=True)
        var = x.var(axis=-1, keepdims=True)
        return (x - mu) / jnp.sqrt(var + 1e-5) * g + b

    def ntanh(x):
        mu = x.mean(axis=-1, keepdims=True)
        var = x.var(axis=-1, keepdims=True)
        return jnp.tanh((x - mu) / jnp.sqrt(var + 1e-5))

    xg = xg_r[...]
    for i in range(3):
        hg = (jnp.dot(xg, wsg_r[i], preferred_element_type=jnp.float32)
              + jnp.dot(m_r[i], wng_r[i], preferred_element_type=jnp.float32)
              + bg_r[i])
        xg = jax.nn.gelu(ln(hg, gg_r[i], beg_r[i]))
    g = ntanh(xg)
    g = jax.nn.gelu(jnp.dot(g, w1_r[...], preferred_element_type=jnp.float32)
                    + b1_r[...])
    g = jax.nn.gelu(jnp.dot(g, w2_r[...], preferred_element_type=jnp.float32)
                    + b2_r[...])
    g = jax.nn.gelu(jnp.dot(g, w3_r[...], preferred_element_type=jnp.float32)
                    + b3_r[...])
    g = ntanh(g)
    g = jnp.dot(g, w4_r[...], preferred_element_type=jnp.float32) + b4_r[...]
    g_out[...] = g

    a = jnp.maximum(g[0:1], 0.0)
    b = jnp.maximum(g[1:2], 0.0)
    num = jnp.sum(jnp.minimum(a, b), axis=-1, keepdims=True)
    den = jnp.sum(jnp.maximum(a, b), axis=-1, keepdims=True) + 1e-8
    ip_out[...] = num / den

    g1, g2 = g[0:1], g[1:2]
    tot = jnp.sum(g1, axis=-1, keepdims=True) + jnp.sum(g2, axis=-1,
                                                        keepdims=True)
    tot2 = (jnp.sum(g1 * g1, axis=-1, keepdims=True)
            + jnp.sum(g2 * g2, axis=-1, keepdims=True))
    mu = tot / (2 * D)
    var = tot2 / (2 * D) - mu * mu
    rstd = 1.0 / jnp.sqrt(var + 1e-5)
    h1 = (g1 - mu) * rstd * lng_r[0] + lnb_r[0]
    h2 = (g2 - mu) * rstd * lng_r[1] + lnb_r[1]
    h = jax.nn.gelu(
        jnp.dot(h1, wp1_r[0], preferred_element_type=jnp.float32)
        + jnp.dot(h2, wp1_r[1], preferred_element_type=jnp.float32)
        + bp1_r[...])
    h = jax.nn.gelu(jnp.dot(h, wp2_r[...], preferred_element_type=jnp.float32)
                    + bp2_r[...])
    h = jax.nn.gelu(jnp.dot(h, wp3_r[...], preferred_element_type=jnp.float32)
                    + bp3_r[...])
    h = ln(h, ln2g_r[...], ln2b_r[...])
    fx_out[...] = (jnp.dot(h, wp4_r[...], preferred_element_type=jnp.float32)
                   + bp4_r[...])


@functools.cache
def _god():
    return pl.pallas_call(
        _god_body,
        out_shape=[jax.ShapeDtypeStruct((2, D), jnp.float32),
                   jax.ShapeDtypeStruct((1, 1), jnp.float32),
                   jax.ShapeDtypeStruct((1, 23), jnp.float32)],
    )


def _ln(x, g, b):
    mu = x.mean(axis=-1, keepdims=True)
    var = x.var(axis=-1, keepdims=True)
    return (x - mu) / jnp.sqrt(var + 1e-5) * g + b


def _normtanh(x):
    mu = x.mean(axis=-1, keepdims=True)
    var = x.var(axis=-1, keepdims=True)
    return jnp.tanh((x - mu) / jnp.sqrt(var + 1e-5))


def _agg_call(x, idx2):
    aggf = _segsum()(x, idx2)
    return jnp.concatenate([aggf[:N], aggf[OUTR:OUTR + N]], axis=0)


def _deg_call(dst2):
    degf = _deg()(dst2)
    return jnp.concatenate([degf[:N, 0:1], degf[OUTR:OUTR + N, 0:1]], axis=0)


def kernel(x1_res, x1_god, x2_res, x2_god, x1_edge_index, x2_edge_index,
           contact_pred_index, params):
    p = params
    padi = jnp.zeros((EPAD - E,), jnp.int32)
    padd = jnp.full((EPAD - E,), DUMMY, jnp.int32)
    src2 = jnp.concatenate([x1_edge_index[0], padi,
                            x2_edge_index[0] + N, padi])
    dst2 = jnp.concatenate([x1_edge_index[1], padd,
                            x2_edge_index[1], padd])
    idx2 = jnp.stack([src2.reshape(-1, CH), dst2.reshape(-1, CH)],
                     axis=1).reshape(-1, CH)

    x = jnp.concatenate([x1_res, x2_res], axis=0)
    inz = x
    xg = jnp.concatenate([x1_god, x2_god], axis=0)

    deg = _deg_call(dst2)
    hs = []
    colsums = []
    for i in range(3):
        agg = _agg_call(x, idx2)
        h, cs = _dense()(
            x, agg, deg,
            p['Wself_res'][i], p['Wnbr_res'][i],
            p['b_res'][i].reshape(1, D), p['g_res'][i].reshape(1, D),
            p['be_res'][i].reshape(1, D))
        colsums.append(cs)
        hs.append(h)
        x = h

    half = NBLK // 2
    m = jnp.stack([
        jnp.stack([cs[:half, 0].sum(axis=0) / N,
                   cs[half:, 0].sum(axis=0) / N])
        for cs in colsums])

    Wc1 = p['Wc1']
    wa = jnp.stack([Wc1[0:D], Wc1[2 * D:3 * D]])
    wb = jnp.stack([Wc1[D:2 * D], Wc1[3 * D:4 * D]])
    u = _jk()(
        hs[0], hs[1], hs[2], inz,
        p['ln_jk_g'].reshape(3, 1, D), p['ln_jk_b'].reshape(3, 1, D),
        p['W_lin'].reshape(3, D, D), p['b_lin'].reshape(1, D),
        wa, wb)

    ci = contact_pred_index
    idx = jnp.concatenate([
        ci[0], ci[1] + N,
        jnp.zeros((IPAD - 2 * P,), jnp.int32)])
    pu = _pair_gather()(u, idx)

    edge_probs = _contact()(
        pu, pu,
        p['bc1'].reshape(1, D),
        p['Wc2'], p['bc2'].reshape(1, D),
        p['Wc3'], p['bc3'].reshape(1, D),
        p['Wc4'].reshape(1, D),
        p['bc4'].reshape(1, 1))

    g, ip, foldx_pred = _god()(
        xg, m,
        p['Wself_god'], p['Wnbr_god'],
        p['b_god'].reshape(3, 1, D), p['g_god'].reshape(3, 1, D),
        p['be_god'].reshape(3, 1, D),
        p['Wg1'], p['bg1'].reshape(1, D),
        p['Wg2'], p['bg2'].reshape(1, D),
        p['Wg3'], p['bg3'].reshape(1, D),
        p['Wg4'], p['bg4'].reshape(1, D),
        p['lnp_g'].reshape(2, 1, D), p['lnp_b'].reshape(2, 1, D),
        p['Wp1'].reshape(2, D, D), p['bp1'].reshape(1, D),
        p['Wp2'], p['bp2'].reshape(1, D),
        p['Wp3'], p['bp3'].reshape(1, D),
        p['lnp2_g'].reshape(1, D), p['lnp2_b'].reshape(1, D),
        p['Wp4'], p['bp4'].reshape(1, 23))

    g1, g2 = g[0:1], g[1:2]
    interaction_prob = ip.reshape((1,))

    return interaction_prob, edge_probs, g1, g2, foldx_pred


# speedup vs baseline: 1.3942x; 1.3942x over previous
import functools

import jax
import jax.numpy as jnp
from jax import lax
from jax.experimental import pallas as pl
from jax.experimental.pallas import tpu as pltpu
from jax.experimental.pallas import tpu_sc as plsc

N = 10000
NN = 2 * N
D = 128
E = 320000
P = 20000
CH = 192
EPAD = 322560
ET = EPAD // 16
NCH = ET // CH
ACC_R = 10112
DUMMY = N
CPO = 632
OUTR = 16 * CPO
IPAD = 40960
IT = IPAD // 32

@functools.cache
def _mesh():
    return plsc.VectorSubcoreMesh(core_axis_name="c", subcore_axis_name="s")


def _zero_fill(buf, rows, width):
    zv = jnp.zeros((16,), jnp.float32)

    def body(i, _):
        for j in range(width // 16):
            buf[i, pl.ds(j * 16, 16)] = zv
        return 0

    lax.fori_loop(0, rows, body, 0)


def _zero_acc(src_buf, acc, s):
    for j in range(4):
        pltpu.sync_copy(src_buf.at[pl.ds(0, 128)],
                        acc.at[pl.ds(s * CPO + j * 128, 128)])
    pltpu.sync_copy(src_buf.at[pl.ds(0, CPO - 512)],
                    acc.at[pl.ds(s * CPO + 512, CPO - 512)])


def _segsum_body(x, src2, dst2, out, acc, sidx, didx, rows, sem):
    c = lax.axis_index("c")
    s = lax.axis_index("s")

    _zero_fill(rows, CH, D)
    _zero_acc(rows, acc, s)

    plsc.subcore_barrier()

    ebase = c * EPAD + s * ET

    def echunk(i, _):
        e0 = ebase + i * CH
        pltpu.sync_copy(src2.at[pl.ds(e0, CH)], sidx)
        pltpu.sync_copy(dst2.at[pl.ds(e0, CH)], didx)
        pltpu.async_copy(x.at[sidx], rows, sem).wait()
        pltpu.sync_copy(rows, acc.at[didx], add=True)
        return 0

    lax.fori_loop(0, NCH, echunk, 0)

    plsc.subcore_barrier()

    pltpu.sync_copy(
        acc.at[pl.ds(s * CPO, CPO)],
        out.at[pl.ds(c * OUTR + s * CPO, CPO)])


@functools.cache
def _segsum():
    return pl.kernel(
    _segsum_body,
    out_type=jax.ShapeDtypeStruct((2 * OUTR, D), jnp.float32),
    mesh=_mesh(),
    scratch_types=[
        pltpu.VMEM_SHARED((ACC_R, D), jnp.float32),
        pltpu.VMEM((CH,), jnp.int32),
        pltpu.VMEM((CH,), jnp.int32),
        pltpu.VMEM((CH, D), jnp.float32),
        pltpu.SemaphoreType.DMA,
    ],
)


def _deg_body(dst2, outdeg, accd, didx, ones, zbufd):
    c = lax.axis_index("c")
    s = lax.axis_index("s")

    ov = jnp.ones((16,), jnp.float32)
    zv = jnp.zeros((16,), jnp.float32)

    def fill(i, _):
        for j in range(D // 16):
            ones[i, pl.ds(j * 16, 16)] = ov
            zbufd[i, pl.ds(j * 16, 16)] = zv
        return 0

    lax.fori_loop(0, CH, fill, 0)
    _zero_acc(zbufd, accd, s)

    plsc.subcore_barrier()

    ebase = c * EPAD + s * ET

    def echunk(i, _):
        pltpu.sync_copy(dst2.at[pl.ds(ebase + i * CH, CH)], didx)
        pltpu.sync_copy(ones, accd.at[didx], add=True)
        return 0

    lax.fori_loop(0, NCH, echunk, 0)

    plsc.subcore_barrier()

    pltpu.sync_copy(
        accd.at[pl.ds(s * CPO, CPO)],
        outdeg.at[pl.ds(c * OUTR + s * CPO, CPO)])


@functools.cache
def _deg():
    return pl.kernel(
    _deg_body,
    out_type=jax.ShapeDtypeStruct((2 * OUTR, D), jnp.float32),
    mesh=_mesh(),
    scratch_types=[
        pltpu.VMEM_SHARED((ACC_R, D), jnp.float32),
        pltpu.VMEM((CH,), jnp.int32),
        pltpu.VMEM((CH, D), jnp.float32),
        pltpu.VMEM((CH, D), jnp.float32),
    ],
)


def _pair_gather_body(table, idx, out, sidx, rows, sem):
    c = lax.axis_index("c")
    s = lax.axis_index("s")
    wid = s * 2 + c
    base = wid * IT

    def chunk(i, _):
        b0 = base + i * CH
        pltpu.sync_copy(idx.at[pl.ds(b0, CH)], sidx)
        pltpu.async_copy(table.at[sidx], rows, sem).wait()
        pltpu.sync_copy(rows, out.at[pl.ds(b0, CH)])
        return 0

    lax.fori_loop(0, IT // CH, chunk, 0)


@functools.cache
def _pair_gather():
    return pl.kernel(
    _pair_gather_body,
    out_type=jax.ShapeDtypeStruct((IPAD, D), jnp.float32),
    mesh=_mesh(),
    scratch_types=[
        pltpu.VMEM((CH,), jnp.int32),
        pltpu.VMEM((CH, D), jnp.float32),
        pltpu.SemaphoreType.DMA,
    ],
)


BR = 400
NBLK = NN // BR


def _dense_body(x_r, agg_r, deg_r, ws_r, wn_r, b_r, g_r, be_r, h_r, cs_r):
    x = x_r[...]
    agg = agg_r[...] / jnp.maximum(deg_r[...], 1.0)
    h = (jnp.dot(x, ws_r[...], preferred_element_type=jnp.float32)
         + jnp.dot(agg, wn_r[...], preferred_element_type=jnp.float32)
         + b_r[...])
    mu = h.mean(axis=-1, keepdims=True)
    var = h.var(axis=-1, keepdims=True)
    h = (h - mu) / jnp.sqrt(var + 1e-5) * g_r[...] + be_r[...]
    h_r[...] = jax.nn.gelu(h)
    cs_r[...] = jnp.broadcast_to(jnp.sum(x, axis=0, keepdims=True)[None],
                                 (1, 8, D))


@functools.cache
def _dense():
    bs = pl.BlockSpec((BR, D), lambda i: (i, 0))
    wfull = pl.BlockSpec((D, D), lambda i: (0, 0))
    vec = pl.BlockSpec((1, D), lambda i: (0, 0))
    return pl.pallas_call(
        _dense_body,
        grid=(NBLK,),
        in_specs=[bs, bs, pl.BlockSpec((BR, 1), lambda i: (i, 0)),
                  wfull, wfull, vec, vec, vec],
        out_specs=[bs, pl.BlockSpec((1, 8, D), lambda i: (i, 0, 0))],
        out_shape=[jax.ShapeDtypeStruct((NN, D), jnp.float32),
                   jax.ShapeDtypeStruct((NBLK, 8, D), jnp.float32)],
    )


def _jk_body(h1_r, h2_r, h3_r, inz_r, g_r, b_r, wl_r, bl_r, wa_r, wb_r, u_r):
    hs = [h1_r[...], h2_r[...], h3_r[...]]
    tot = sum(h.sum(axis=-1, keepdims=True) for h in hs)
    tot2 = sum((h * h).sum(axis=-1, keepdims=True) for h in hs)
    mu = tot / (3 * D)
    var = tot2 / (3 * D) - mu * mu
    rstd = 1.0 / jnp.sqrt(var + 1e-5)
    z = bl_r[...]
    for j, h in enumerate(hs):
        lnh = (h - mu) * rstd * g_r[j] + b_r[j]
        z = z + jnp.dot(lnh, wl_r[j], preferred_element_type=jnp.float32)
    u_r[...] = (jnp.dot(inz_r[...], wa_r[0],
                        preferred_element_type=jnp.float32)
                + jnp.dot(z, wb_r[0], preferred_element_type=jnp.float32))


@functools.cache
def _jk():
    bs = pl.BlockSpec((BR, D), lambda i: (i, 0))
    w3 = pl.BlockSpec((3, D, D), lambda i: (0, 0, 0))
    v3 = pl.BlockSpec((3, 1, D), lambda i: (0, 0, 0))
    vec = pl.BlockSpec((1, D), lambda i: (0, 0))
    wg = pl.BlockSpec((1, D, D), lambda i: (i // (NBLK // 2), 0, 0))
    return pl.pallas_call(
        _jk_body,
        grid=(NBLK,),
        in_specs=[bs, bs, bs, bs, v3, v3, w3, vec, wg, wg],
        out_specs=bs,
        out_shape=jax.ShapeDtypeStruct((NN, D), jnp.float32),
    )


def _contact_body2(p1_r, p2_r, bc1_r, w2_r, b2_r, w3_r, b3_r, w4_r, bc4_r,
                   ep_r):
    h = jax.nn.gelu(p1_r[...] + p2_r[...] + bc1_r[...])
    h = jax.nn.gelu(jnp.dot(h, w2_r[...], preferred_element_type=jnp.float32)
                    + b2_r[...])
    h = jax.nn.gelu(jnp.dot(h, w3_r[...], preferred_element_type=jnp.float32)
                    + b3_r[...])
    mu = h.mean(axis=-1, keepdims=True)
    var = h.var(axis=-1, keepdims=True)
    h = jnp.tanh((h - mu) / jnp.sqrt(var + 1e-5))
    logit = jnp.sum(h * w4_r[...], axis=-1, keepdims=True) + bc4_r[0, 0]
    ep_r[...] = jax.nn.sigmoid(logit)


PBLK = P // BR


@functools.cache
def _contact():
    bs = pl.BlockSpec((BR, D), lambda i: (i, 0))
    wfull = pl.BlockSpec((D, D), lambda i: (0, 0))
    vec = pl.BlockSpec((1, D), lambda i: (0, 0))
    one = pl.BlockSpec((1, 1), lambda i: (0, 0))
    bs2 = pl.BlockSpec((BR, D), lambda i: (i + PBLK, 0))
    return pl.pallas_call(
        _contact_body2,
        grid=(PBLK,),
        in_specs=[bs, bs2, vec, wfull, vec, wfull, vec, vec, one],
        out_specs=pl.BlockSpec((BR, 1), lambda i: (i, 0)),
        out_shape=jax.ShapeDtypeStruct((P, 1), jnp.float32),
    )


def _god_body(xg_r, m_r, wsg_r, wng_r, bg_r, gg_r, beg_r,
              w1_r, b1_r, w2_r, b2_r, w3_r, b3_r, w4_r, b4_r,
              lng_r, lnb_r, wp1_r, bp1_r, wp2_r, bp2_r, wp3_r, bp3_r,
              ln2g_r, ln2b_r, wp4_r, bp4_r,
              g_out, ip_out, fx_out):
    def ln(x, g, b):
        mu = x.mean(axis=-1, keepdims=True)
        var = x.var(axis=-1, keepdims=True)
        return (x - mu) / jnp.sqrt(var + 1e-5) * g + b

    def ntanh(x):
        mu = x.mean(axis=-1, keepdims=True)
        var = x.var(axis=-1, keepdims=True)
        return jnp.tanh((x - mu) / jnp.sqrt(var + 1e-5))

    xg = xg_r[...]
    for i in range(3):
        hg = (jnp.dot(xg, wsg_r[i], preferred_element_type=jnp.float32)
              + jnp.dot(m_r[i], wng_r[i], preferred_element_type=jnp.float32)
              + bg_r[i])
        xg = jax.nn.gelu(ln(hg, gg_r[i], beg_r[i]))
    g = ntanh(xg)
    g = jax.nn.gelu(jnp.dot(g, w1_r[...], preferred_element_type=jnp.float32)
                    + b1_r[...])
    g = jax.nn.gelu(jnp.dot(g, w2_r[...], preferred_element_type=jnp.float32)
                    + b2_r[...])
    g = jax.nn.gelu(jnp.dot(g, w3_r[...], preferred_element_type=jnp.float32)
                    + b3_r[...])
    g = ntanh(g)
    g = jnp.dot(g, w4_r[...], preferred_element_type=jnp.float32) + b4_r[...]
    g_out[...] = g

    a = jnp.maximum(g[0:1], 0.0)
    b = jnp.maximum(g[1:2], 0.0)
    num = jnp.sum(jnp.minimum(a, b), axis=-1, keepdims=True)
    den = jnp.sum(jnp.maximum(a, b), axis=-1, keepdims=True) + 1e-8
    ip_out[...] = num / den

    g1, g2 = g[0:1], g[1:2]
    tot = jnp.sum(g1, axis=-1, keepdims=True) + jnp.sum(g2, axis=-1,
                                                        keepdims=True)
    tot2 = (jnp.sum(g1 * g1, axis=-1, keepdims=True)
            + jnp.sum(g2 * g2, axis=-1, keepdims=True))
    mu = tot / (2 * D)
    var = tot2 / (2 * D) - mu * mu
    rstd = 1.0 / jnp.sqrt(var + 1e-5)
    h1 = (g1 - mu) * rstd * lng_r[0] + lnb_r[0]
    h2 = (g2 - mu) * rstd * lng_r[1] + lnb_r[1]
    h = jax.nn.gelu(
        jnp.dot(h1, wp1_r[0], preferred_element_type=jnp.float32)
        + jnp.dot(h2, wp1_r[1], preferred_element_type=jnp.float32)
        + bp1_r[...])
    h = jax.nn.gelu(jnp.dot(h, wp2_r[...], preferred_element_type=jnp.float32)
                    + bp2_r[...])
    h = jax.nn.gelu(jnp.dot(h, wp3_r[...], preferred_element_type=jnp.float32)
                    + bp3_r[...])
    h = ln(h, ln2g_r[...], ln2b_r[...])
    fx_out[...] = (jnp.dot(h, wp4_r[...], preferred_element_type=jnp.float32)
                   + bp4_r[...])


@functools.cache
def _god():
    return pl.pallas_call(
        _god_body,
        out_shape=[jax.ShapeDtypeStruct((2, D), jnp.float32),
                   jax.ShapeDtypeStruct((1, 1), jnp.float32),
                   jax.ShapeDtypeStruct((1, 23), jnp.float32)],
    )


def _ln(x, g, b):
    mu = x.mean(axis=-1, keepdims=True)
    var = x.var(axis=-1, keepdims=True)
    return (x - mu) / jnp.sqrt(var + 1e-5) * g + b


def _normtanh(x):
    mu = x.mean(axis=-1, keepdims=True)
    var = x.var(axis=-1, keepdims=True)
    return jnp.tanh((x - mu) / jnp.sqrt(var + 1e-5))


def _agg_call(x, src2, dst2):
    aggf = _segsum()(x, src2, dst2)
    return jnp.concatenate([aggf[:N], aggf[OUTR:OUTR + N]], axis=0)


def _deg_call(dst2):
    degf = _deg()(dst2)
    return jnp.concatenate([degf[:N, 0:1], degf[OUTR:OUTR + N, 0:1]], axis=0)


def kernel(x1_res, x1_god, x2_res, x2_god, x1_edge_index, x2_edge_index,
           contact_pred_index, params):
    p = params
    padi = jnp.zeros((EPAD - E,), jnp.int32)
    padd = jnp.full((EPAD - E,), DUMMY, jnp.int32)
    src2 = jnp.concatenate([x1_edge_index[0], padi,
                            x2_edge_index[0] + N, padi])
    dst2 = jnp.concatenate([x1_edge_index[1], padd,
                            x2_edge_index[1], padd])

    x = jnp.concatenate([x1_res, x2_res], axis=0)
    inz = x
    xg = jnp.concatenate([x1_god, x2_god], axis=0)

    deg = _deg_call(dst2)
    hs = []
    colsums = []
    for i in range(3):
        agg = _agg_call(x, src2, dst2)
        h, cs = _dense()(
            x, agg, deg,
            p['Wself_res'][i], p['Wnbr_res'][i],
            p['b_res'][i].reshape(1, D), p['g_res'][i].reshape(1, D),
            p['be_res'][i].reshape(1, D))
        colsums.append(cs)
        hs.append(h)
        x = h

    half = NBLK // 2
    m = jnp.stack([
        jnp.stack([cs[:half, 0].sum(axis=0) / N,
                   cs[half:, 0].sum(axis=0) / N])
        for cs in colsums])

    Wc1 = p['Wc1']
    wa = jnp.stack([Wc1[0:D], Wc1[2 * D:3 * D]])
    wb = jnp.stack([Wc1[D:2 * D], Wc1[3 * D:4 * D]])
    u = _jk()(
        hs[0], hs[1], hs[2], inz,
        p['ln_jk_g'].reshape(3, 1, D), p['ln_jk_b'].reshape(3, 1, D),
        p['W_lin'].reshape(3, D, D), p['b_lin'].reshape(1, D),
        wa, wb)

    ci = contact_pred_index
    idx = jnp.concatenate([
        ci[0], ci[1] + N,
        jnp.zeros((IPAD - 2 * P,), jnp.int32)])
    pu = _pair_gather()(u, idx)

    edge_probs = _contact()(
        pu, pu,
        p['bc1'].reshape(1, D),
        p['Wc2'], p['bc2'].reshape(1, D),
        p['Wc3'], p['bc3'].reshape(1, D),
        p['Wc4'].reshape(1, D),
        p['bc4'].reshape(1, 1))

    g, ip, foldx_pred = _god()(
        xg, m,
        p['Wself_god'], p['Wnbr_god'],
        p['b_god'].reshape(3, 1, D), p['g_god'].reshape(3, 1, D),
        p['be_god'].reshape(3, 1, D),
        p['Wg1'], p['bg1'].reshape(1, D),
        p['Wg2'], p['bg2'].reshape(1, D),
        p['Wg3'], p['bg3'].reshape(1, D),
        p['Wg4'], p['bg4'].reshape(1, D),
        p['lnp_g'].reshape(2, 1, D), p['lnp_b'].reshape(2, 1, D),
        p['Wp1'].reshape(2, D, D), p['bp1'].reshape(1, D),
        p['Wp2'], p['bp2'].reshape(1, D),
        p['Wp3'], p['bp3'].reshape(1, D),
        p['lnp2_g'].reshape(1, D), p['lnp2_b'].reshape(1, D),
        p['Wp4'], p['bp4'].reshape(1, 23))

    g1, g2 = g[0:1], g[1:2]
    interaction_prob = ip.reshape((1,))

    return interaction_prob, edge_probs, g1, g2, foldx_pred
